# baseline (speedup 1.0000x reference)
import functools

import jax
import jax.numpy as jnp
from jax import lax
from jax.experimental import pallas as pl
from jax.experimental.pallas import tpu as pltpu
from jax.experimental.pallas import tpu_sc as plsc

_MARGIN = 0.5
_EPS = 1e-8
_NBUF = 2
_PIECE = 256


def _normalize_body(e_ref, o_ref):
    e = e_ref[...]
    s = jnp.sum(e * e, axis=1, keepdims=True)
    n = jnp.maximum(jnp.sqrt(s), _EPS)
    eh = (e / n).astype(jnp.bfloat16)
    d2 = e.shape[1] // 2
    lo = jax.lax.bitcast_convert_type(
        eh[:, :d2], jnp.uint16).astype(jnp.uint32)
    hi = jax.lax.bitcast_convert_type(
        eh[:, d2:], jnp.uint16).astype(jnp.uint32)
    o_ref[...] = jax.lax.bitcast_convert_type(
        lo | (hi << 16), jnp.float32)


def _make_sc_loss(n_pairs, n_rows_pad, d_words, chunk, cpw, nc, ns):
    nw = nc * ns
    mesh = plsc.VectorSubcoreMesh(core_axis_name="c", subcore_axis_name="s")
    groups = chunk // 16
    dchunks = d_words // 16
    ppw = cpw * chunk
    tail_w = n_pairs // ppw
    tail_rows = n_pairs - tail_w * ppw
    rpt = (n_rows_pad // ns) // 8 * 8
    rpt_last = n_rows_pad - rpt * (ns - 1)

    row_bufs = [pltpu.VMEM((2 * chunk, d_words), jnp.float32)
                for _ in range(_NBUF)]
    sems = [pltpu.SemaphoreType.DMA for _ in range(_NBUF)]

    @functools.partial(
        pl.kernel,
        mesh=mesh,
        compiler_params=pltpu.CompilerParams(
            use_tc_tiling_on_sc=False, needs_layout_passes=False),
        out_type=jax.ShapeDtypeStruct((nw, 8, 16), jnp.float32),
        scratch_types=[
            pltpu.VMEM_SHARED((n_rows_pad, d_words), jnp.float32),
            pltpu.VMEM((cpw, 2 * chunk), jnp.int32),
            pltpu.VMEM((cpw, chunk), jnp.int32),
            pltpu.VMEM((16, 16), jnp.float32),
            pltpu.VMEM((8, 16), jnp.float32),
        ] + row_bufs + sems,
    )
    def sc_loss(emb, idxc, labels, out, table_s, idx_v, lab_v,
                dred, acc_v, *bufs_and_sems):
        rc = bufs_and_sems[0:_NBUF]
        sc = bufs_and_sems[_NBUF:2 * _NBUF]
        cid = lax.axis_index("c")
        sid = lax.axis_index("s")
        wid = sid * nc + cid
        lanes = lax.broadcasted_iota(jnp.int32, (16,), 0)
        base = wid * cpw
        pltpu.sync_copy(idxc.at[pl.ds(base, cpw)], idx_v)
        pltpu.sync_copy(labels.at[pl.ds(base, cpw)], lab_v)

        trow = sid * rpt

        @pl.when(sid < ns - 1)
        def _():
            pltpu.sync_copy(emb.at[pl.ds(trow, rpt)],
                            table_s.at[pl.ds(trow, rpt)])

        @pl.when(sid == ns - 1)
        def _():
            pltpu.sync_copy(emb.at[pl.ds(trow, rpt_last)],
                            table_s.at[pl.ds(trow, rpt_last)])

        plsc.subcore_barrier()

        zero16 = jnp.zeros((16,), jnp.float32)

        def issue(j, b):
            pltpu.async_copy(table_s.at[idx_v.at[j]], rc[b], sc[b])

        def wait(j, b):
            pltpu.make_async_copy(
                table_s.at[idx_v.at[j]], rc[b], sc[b]).wait()

        def compute(j, b, acc):
            ra = rb = rc[b]

            def group_body(g, acc):
                for p16 in range(16):
                    p = g * 16 + p16
                    a = zero16
                    bb = zero16
                    for t in range(dchunks):
                        w1 = plsc.bitcast(ra[p, pl.ds(16 * t, 16)],
                                          jnp.bfloat16)
                        w2 = plsc.bitcast(rb[chunk + p, pl.ds(16 * t, 16)],
                                          jnp.bfloat16)
                        u1, v1 = plsc.unpack(
                            w1, format=plsc.PackFormat.INTERLEAVED)
                        u2, v2 = plsc.unpack(
                            w2, format=plsc.PackFormat.INTERLEAVED)
                        a = a + u1 * u2
                        bb = bb + v1 * v2
                    dred[p16, :] = a + bb
                dots = plsc.load_gather(
                    dred, [lanes, jnp.zeros((16,), jnp.int32)])
                for c in range(1, 16):
                    dots = dots + plsc.load_gather(
                        dred, [lanes, jnp.full((16,), c, jnp.int32)])
                l = lab_v[j, pl.ds(g * 16, 16)].astype(jnp.float32)
                t = 0.5 * (dots + 1.0)
                clamped = jnp.maximum(_MARGIN - t, 0.0)
                loss = (1.0 - l) * t * t + l * clamped * clamped
                return acc + loss

            return lax.fori_loop(0, groups, group_body, acc)

        for b in range(_NBUF):
            issue(b, b)

        def ring_body(jj, acc):
            for b in range(_NBUF):
                j = _NBUF * jj + b
                wait(j, b)
                acc = compute(j, b, acc)

                @pl.when(j + _NBUF < cpw)
                def _():
                    issue(j + _NBUF, b)

            return acc

        acc = lax.fori_loop(0, cpw // _NBUF, ring_body, zero16)
        acc_v[0, :] = acc
        for r in range(1, 8):
            acc_v[r, :] = zero16
        pltpu.sync_copy(acc_v, out.at[wid])

    return sc_loss


def kernel(embeddings, pairs):
    n_nodes, d_feat = embeddings.shape
    n_pairs = pairs.shape[0]
    info = plsc.get_sparse_core_info()
    nc, ns = info.num_cores, info.num_subcores
    nw = nc * ns
    chunk = 64
    per = nw * chunk
    cpw = -(-n_pairs // per)
    cpw = -(-cpw // 8) * 8
    np_pad = cpw * per

    rblk = n_nodes // 5
    d_words = d_feat // 2
    norm = pl.pallas_call(
        _normalize_body,
        out_shape=jax.ShapeDtypeStruct((n_nodes, d_words), jnp.float32),
        grid=(5,),
        in_specs=[pl.BlockSpec((rblk, d_feat), lambda i: (i, 0))],
        out_specs=pl.BlockSpec((rblk, d_words), lambda i: (i, 0)),
    )(embeddings)
    n_rows_pad = n_nodes

    pad = np_pad - n_pairs
    pad_idx = jnp.arange(pad, dtype=jnp.int32) % jnp.int32(n_nodes)
    idx1 = jnp.concatenate(
        [pairs[:, 0], pad_idx]).reshape(nw * cpw, chunk)
    idx2 = jnp.concatenate(
        [pairs[:, 1], pad_idx]).reshape(nw * cpw, chunk)
    idxc = jnp.concatenate([idx1, idx2], axis=1)
    lab = jnp.concatenate(
        [pairs[:, 2], jnp.ones((pad,), jnp.int32)]).reshape(nw * cpw, chunk)

    sc_loss = _make_sc_loss(n_pairs, n_rows_pad, d_words, chunk, cpw, nc, ns)
    partials = sc_loss(norm, idxc, lab)
    return jnp.sum(partials) / jnp.float32(n_pairs)


# speedup vs baseline: 1.0009x; 1.0009x over previous
import functools

import jax
import jax.numpy as jnp
from jax import lax
from jax.experimental import pallas as pl
from jax.experimental.pallas import tpu as pltpu
from jax.experimental.pallas import tpu_sc as plsc

_MARGIN = 0.5
_EPS = 1e-8
_NBUF = 2


def _normalize_body(e_ref, o_ref):
    e = e_ref[...]
    s = jnp.sum(e * e, axis=1, keepdims=True)
    n = jnp.maximum(jnp.sqrt(s), _EPS)
    eh = (e / n).astype(jnp.bfloat16)
    d2 = e.shape[1] // 2
    lo = jax.lax.bitcast_convert_type(
        eh[:, :d2], jnp.uint16).astype(jnp.uint32)
    hi = jax.lax.bitcast_convert_type(
        eh[:, d2:], jnp.uint16).astype(jnp.uint32)
    o_ref[...] = jax.lax.bitcast_convert_type(
        lo | (hi << 16), jnp.float32)


def _make_sc_loss(n_rows_pad, d_words, chunk, cpw, nc, ns):
    nw = nc * ns
    mesh = plsc.VectorSubcoreMesh(core_axis_name="c", subcore_axis_name="s")
    groups = chunk // 16
    dchunks = d_words // 16
    rpt = (n_rows_pad // ns) // 8 * 8
    rpt_last = n_rows_pad - rpt * (ns - 1)

    row_bufs = [pltpu.VMEM((2 * chunk, d_words), jnp.float32)
                for _ in range(_NBUF)]
    sems = [pltpu.SemaphoreType.DMA for _ in range(_NBUF)]

    @functools.partial(
        pl.kernel,
        mesh=mesh,
        compiler_params=pltpu.CompilerParams(
            use_tc_tiling_on_sc=False, needs_layout_passes=False),
        out_type=jax.ShapeDtypeStruct((nw, 8, 16), jnp.float32),
        scratch_types=[
            pltpu.VMEM_SHARED((n_rows_pad, d_words), jnp.float32),
            pltpu.VMEM((cpw, 2 * chunk), jnp.int32),
            pltpu.VMEM((cpw, chunk), jnp.int32),
            pltpu.VMEM((16, 16), jnp.float32),
            pltpu.VMEM((8, 16), jnp.float32),
        ] + row_bufs + sems,
    )
    def sc_loss(emb, idxc, labels, out, table_s, idx_v, lab_v,
                dred, acc_v, *bufs_and_sems):
        rc = bufs_and_sems[0:_NBUF]
        sc = bufs_and_sems[_NBUF:2 * _NBUF]
        cid = lax.axis_index("c")
        sid = lax.axis_index("s")
        wid = sid * nc + cid
        lanes = lax.broadcasted_iota(jnp.int32, (16,), 0)
        base = wid * cpw
        pltpu.sync_copy(idxc.at[pl.ds(base, cpw)], idx_v)
        pltpu.sync_copy(labels.at[pl.ds(base, cpw)], lab_v)

        trow = sid * rpt

        @pl.when(sid < ns - 1)
        def _():
            pltpu.sync_copy(emb.at[pl.ds(trow, rpt)],
                            table_s.at[pl.ds(trow, rpt)])

        @pl.when(sid == ns - 1)
        def _():
            pltpu.sync_copy(emb.at[pl.ds(trow, rpt_last)],
                            table_s.at[pl.ds(trow, rpt_last)])

        plsc.subcore_barrier()

        zero16 = jnp.zeros((16,), jnp.float32)

        def issue(j, b):
            pltpu.async_copy(table_s.at[idx_v.at[j]], rc[b], sc[b])

        def wait(j, b):
            pltpu.make_async_copy(
                table_s.at[idx_v.at[j]], rc[b], sc[b]).wait()

        def compute(j, b, acc):
            ra = rb = rc[b]

            def group_body(g, acc):
                for p16 in range(16):
                    p = g * 16 + p16
                    a = zero16
                    bb = zero16
                    for t in range(dchunks):
                        w1 = plsc.bitcast(ra[p, pl.ds(16 * t, 16)],
                                          jnp.bfloat16)
                        w2 = plsc.bitcast(rb[chunk + p, pl.ds(16 * t, 16)],
                                          jnp.bfloat16)
                        u1, v1 = plsc.unpack(
                            w1, format=plsc.PackFormat.INTERLEAVED)
                        u2, v2 = plsc.unpack(
                            w2, format=plsc.PackFormat.INTERLEAVED)
                        a = a + u1 * u2
                        bb = bb + v1 * v2
                    dred[p16, :] = a + bb
                dots = plsc.load_gather(
                    dred, [lanes, jnp.zeros((16,), jnp.int32)])
                for c in range(1, 16):
                    dots = dots + plsc.load_gather(
                        dred, [lanes, jnp.full((16,), c, jnp.int32)])
                l = lab_v[j, pl.ds(g * 16, 16)].astype(jnp.float32)
                t = 0.5 * (dots + 1.0)
                clamped = jnp.maximum(_MARGIN - t, 0.0)
                loss = (1.0 - l) * t * t + l * clamped * clamped
                return acc + loss

            return lax.fori_loop(0, groups, group_body, acc)

        for b in range(_NBUF):
            issue(b, b)

        def ring_body(jj, acc):
            for b in range(_NBUF):
                j = _NBUF * jj + b
                wait(j, b)
                acc = compute(j, b, acc)

                @pl.when(j + _NBUF < cpw)
                def _():
                    issue(j + _NBUF, b)

            return acc

        acc = lax.fori_loop(0, cpw // _NBUF, ring_body, zero16)
        acc_v[0, :] = acc
        for r in range(1, 8):
            acc_v[r, :] = zero16
        pltpu.sync_copy(acc_v, out.at[wid])

    return sc_loss


def kernel(embeddings, pairs):
    n_nodes, d_feat = embeddings.shape
    n_pairs = pairs.shape[0]
    info = plsc.get_sparse_core_info()
    nc, ns = info.num_cores, info.num_subcores
    nw = nc * ns
    chunk = 64
    per = nw * chunk
    cpw = -(-n_pairs // per)
    cpw = -(-cpw // 8) * 8
    np_pad = cpw * per

    rblk = n_nodes // 5
    d_words = d_feat // 2
    norm = pl.pallas_call(
        _normalize_body,
        out_shape=jax.ShapeDtypeStruct((n_nodes, d_words), jnp.float32),
        grid=(5,),
        in_specs=[pl.BlockSpec((rblk, d_feat), lambda i: (i, 0))],
        out_specs=pl.BlockSpec((rblk, d_words), lambda i: (i, 0)),
    )(embeddings)
    pad = np_pad - n_pairs
    pad_idx = jnp.arange(pad, dtype=jnp.int32) % jnp.int32(n_nodes)
    idx1 = jnp.concatenate(
        [pairs[:, 0], pad_idx]).reshape(nw * cpw, chunk)
    idx2 = jnp.concatenate(
        [pairs[:, 1], pad_idx]).reshape(nw * cpw, chunk)
    idxc = jnp.concatenate([idx1, idx2], axis=1)
    lab = jnp.concatenate(
        [pairs[:, 2], jnp.ones((pad,), jnp.int32)]).reshape(nw * cpw, chunk)

    sc_loss = _make_sc_loss(n_nodes, d_words, chunk, cpw, nc, ns)
    partials = sc_loss(norm, idxc, lab)
    return jnp.sum(partials) / jnp.float32(n_pairs)
